# Initial kernel scaffold; baseline (speedup 1.0000x reference)
import jax
import jax.numpy as jnp
from jax.experimental import pallas as pl


def kernel(user_emb, item_emb, adj_values, adj_indices):
    raise NotImplementedError("write your pallas kernel here")


# speedup vs baseline: 5.7559x; 5.7559x over previous
import functools

import jax
import jax.numpy as jnp
from jax import lax
from jax.experimental import pallas as pl
from jax.experimental.pallas import tpu as pltpu
from jax.experimental.pallas import tpu_sc as plsc

USER_N = 5000
ITEM_N = 5000
N = USER_N + ITEM_N
NNZ = 320000
EMB = 128
NLAYERS = 3

NC = 2
NS = 16
NW = NC * NS
EPT = NNZ // NW
CHUNK = 80
NCHUNK = EPT // CHUNK
CBLK = 25
NBLK = NCHUNK // CBLK
STRIPE = 624
TAIL0 = N - NS * STRIPE
TAIL_OFF = NS * STRIPE

_mesh = plsc.VectorSubcoreMesh(
    core_axis_name="c", subcore_axis_name="s", num_cores=NC, num_subcores=NS
)


def _sc_layer_body(ego, rows3, cols3, vals3, zeros, part0, part1,
                   colv, rowv, valv, gbuf, acc, sem):
    c = lax.axis_index("c")
    s = lax.axis_index("s")
    wid = c * NS + s

    row0 = s * STRIPE
    pltpu.sync_copy(zeros.at[pl.ds(row0, STRIPE)], acc.at[pl.ds(row0, STRIPE)])

    @pl.when(s == 0)
    def _():
        pltpu.sync_copy(zeros.at[pl.ds(TAIL_OFF, TAIL0)],
                        acc.at[pl.ds(TAIL_OFF, TAIL0)])

    plsc.subcore_barrier()

    for b in range(NBLK):
        pltpu.sync_copy(rows3.at[wid, b], rowv)
        pltpu.sync_copy(cols3.at[wid, b], colv)
        pltpu.sync_copy(vals3.at[wid, b], valv)

        def chunk_body(j, carry):
            pltpu.async_copy(ego.at[colv.at[j]], gbuf, sem).wait()

            def group_body(g, carry2):
                vals16 = valv[j, pl.ds(g * 16, 16)]
                dnums = lax.GatherDimensionNumbers(
                    offset_dims=(), collapsed_slice_dims=(0,),
                    start_index_map=(0,))
                for lane in range(16):
                    v = lax.gather(vals16, jnp.full((16, 1), lane, jnp.int32),
                                   dnums, slice_sizes=(1,),
                                   mode=lax.GatherScatterMode.PROMISE_IN_BOUNDS)
                    e = g * 16 + lane
                    for k in range(EMB // 16):
                        gbuf[e, pl.ds(k * 16, 16)] = (
                            gbuf[e, pl.ds(k * 16, 16)] * v)
                return carry2

            lax.fori_loop(0, CHUNK // 16, group_body, 0)

            pltpu.sync_copy(gbuf, acc.at[rowv.at[j]], add=True)
            return carry

        lax.fori_loop(0, CBLK, chunk_body, 0)
    plsc.subcore_barrier()

    @pl.when(c == 0)
    def _():
        pltpu.sync_copy(acc.at[pl.ds(row0, STRIPE)],
                        part0.at[pl.ds(row0, STRIPE)])

        @pl.when(s == 0)
        def _():
            pltpu.sync_copy(acc.at[pl.ds(TAIL_OFF, TAIL0)],
                            part0.at[pl.ds(TAIL_OFF, TAIL0)])

    @pl.when(c == 1)
    def _():
        pltpu.sync_copy(acc.at[pl.ds(row0, STRIPE)],
                        part1.at[pl.ds(row0, STRIPE)])

        @pl.when(s == 0)
        def _():
            pltpu.sync_copy(acc.at[pl.ds(TAIL_OFF, TAIL0)],
                            part1.at[pl.ds(TAIL_OFF, TAIL0)])


_sc_layer = functools.partial(
    pl.kernel,
    out_type=(
        jax.ShapeDtypeStruct((N, EMB), jnp.float32),
        jax.ShapeDtypeStruct((N, EMB), jnp.float32),
    ),
    mesh=_mesh,
    scratch_types=[
        pltpu.VMEM((CBLK, CHUNK), jnp.int32),
        pltpu.VMEM((CBLK, CHUNK), jnp.int32),
        pltpu.VMEM((CBLK, CHUNK), jnp.float32),
        pltpu.VMEM((CHUNK, EMB), jnp.float32),
        pltpu.VMEM_SHARED((N, EMB), jnp.float32),
        pltpu.SemaphoreType.DMA,
    ],
)(_sc_layer_body)


_BLK = 1000


def _add2_body(a_ref, b_ref, o_ref):
    o_ref[...] = a_ref[...] + b_ref[...]


def _combine(a, b):
    return pl.pallas_call(
        _add2_body,
        grid=(N // _BLK,),
        in_specs=[pl.BlockSpec((_BLK, EMB), lambda i: (i, 0))] * 2,
        out_specs=pl.BlockSpec((_BLK, EMB), lambda i: (i, 0)),
        out_shape=jax.ShapeDtypeStruct((N, EMB), jnp.float32),
    )(a, b)


def _mean_body(e1_ref, e2_ref, p0_ref, p1_ref, o_ref):
    o_ref[...] = (e1_ref[...] + e2_ref[...] + p0_ref[...] + p1_ref[...]) * (
        1.0 / NLAYERS
    )


def _final_mean(e1, e2, p0, p1):
    return pl.pallas_call(
        _mean_body,
        grid=(N // _BLK,),
        in_specs=[pl.BlockSpec((_BLK, EMB), lambda i: (i, 0))] * 4,
        out_specs=pl.BlockSpec((_BLK, EMB), lambda i: (i, 0)),
        out_shape=jax.ShapeDtypeStruct((N, EMB), jnp.float32),
    )(e1, e2, p0, p1)


def kernel(user_emb, item_emb, adj_values, adj_indices):
    ego = jnp.concatenate([user_emb, item_emb], axis=0)
    rows3 = adj_indices[0].reshape(NW, NBLK, CBLK, CHUNK)
    cols3 = adj_indices[1].reshape(NW, NBLK, CBLK, CHUNK)
    vals3 = adj_values.reshape(NW, NBLK, CBLK, CHUNK)
    zeros = jnp.zeros((N, EMB), jnp.float32)

    p0, p1 = _sc_layer(ego, rows3, cols3, vals3, zeros)
    e1 = _combine(p0, p1)
    p0, p1 = _sc_layer(e1, rows3, cols3, vals3, zeros)
    e2 = _combine(p0, p1)
    p0, p1 = _sc_layer(e2, rows3, cols3, vals3, zeros)
    out = _final_mean(e1, e2, p0, p1)
    return out[:USER_N], out[USER_N:]
